# baseline (speedup 1.0000x reference)
import functools

import jax
import jax.numpy as jnp
from jax import lax
from jax.experimental import pallas as pl
from jax.experimental.pallas import tpu as pltpu
from jax.experimental.pallas import tpu_sc as plsc

_NUM_EMB = 8192
_DIM = 32
_ROW_BLOCK = 256
_COMMIT = 0.5


def _vq_body(z_ref, cb_ref, idx_ref, acc_ref, cbb_ref, cbsq_ref):
    i = pl.program_id(0)
    z = z_ref[...]

    @pl.when(i == 0)
    def _prep():
        cb = cb_ref[...]
        cbb_ref[...] = cb.astype(jnp.bfloat16) * jnp.bfloat16(-2.0)
        cbsq_ref[...] = jnp.sum(cb * cb, axis=1)[None, :]
        acc_ref[...] = jnp.zeros((1, 1), jnp.float32)

    z_sq = jnp.sum(z * z, axis=1, keepdims=True)
    cb_sq = cbsq_ref[...]
    s2 = jax.lax.dot_general(
        z.astype(jnp.bfloat16), cbb_ref[...],
        (((1,), (1,)), ((), ())),
        preferred_element_type=jnp.float32)
    dist = (z_sq + cb_sq) + s2
    half = _NUM_EMB // 2
    d0 = dist[:, :half]
    d1 = dist[:, half:]
    m0 = jnp.min(d0, axis=1, keepdims=True)
    m1 = jnp.min(d1, axis=1, keepdims=True)
    fhalf = jnp.float32(half)
    colf = jax.lax.broadcasted_iota(
        jnp.int32, (1, half), 1).astype(jnp.float32)
    i0f = jnp.min(jnp.where(d0 == m0, colf, fhalf), axis=1)
    i1f = jnp.min(jnp.where(d1 == m1, colf, fhalf), axis=1) + fhalf
    m0_bf = m0[:, 0].astype(jnp.bfloat16).astype(jnp.float32)
    use1 = m1[:, 0] < m0_bf
    idx_ref[0, 0, :] = jnp.where(use1, i1f, i0f).astype(jnp.int32)

    m_sel = jnp.where(use1, m1[:, 0], m0[:, 0])
    acc_ref[...] += jnp.sum(m_sel).reshape(1, 1)


def _argmin_distances(flat_z, codebook):
    n = flat_z.shape[0]
    nblocks = n // _ROW_BLOCK
    idx3, acc = pl.pallas_call(
        _vq_body,
        grid=(nblocks,),
        in_specs=[
            pl.BlockSpec((_ROW_BLOCK, _DIM), lambda i: (i, 0)),
            pl.BlockSpec((_NUM_EMB, _DIM), lambda i: (0, 0)),
        ],
        out_specs=[
            pl.BlockSpec((1, 1, _ROW_BLOCK), lambda i: (i, 0, 0)),
            pl.BlockSpec((1, 1), lambda i: (0, 0)),
        ],
        out_shape=[
            jax.ShapeDtypeStruct((nblocks, 1, _ROW_BLOCK), jnp.int32),
            jax.ShapeDtypeStruct((1, 1), jnp.float32),
        ],
        scratch_shapes=[
            pltpu.VMEM((_NUM_EMB, _DIM), jnp.bfloat16),
            pltpu.VMEM((1, _NUM_EMB), jnp.float32),
        ],
        compiler_params=pltpu.CompilerParams(
            dimension_semantics=("arbitrary",)),
    )(flat_z, codebook)
    return idx3.reshape(n), acc


def _sc_gather(codebook, indices, n):
    info = plsc.get_sparse_core_info()
    nw = info.num_cores * info.num_subcores
    b_per_w = n // nw
    mesh = plsc.VectorSubcoreMesh(core_axis_name="c", subcore_axis_name="s")

    @functools.partial(
        pl.kernel, mesh=mesh,
        out_type=jax.ShapeDtypeStruct((n, _DIM), jnp.float32),
        scratch_types=[
            pltpu.VMEM((b_per_w,), jnp.int32),
            pltpu.VMEM((b_per_w, _DIM), jnp.float32),
            pltpu.SemaphoreType.DMA,
        ],
        compiler_params=pltpu.CompilerParams(use_tc_tiling_on_sc=False),
    )
    def gather(table_hbm, idx_hbm, out_hbm, idx_v, rows_v, sem):
        wid = lax.axis_index("s") * info.num_cores + lax.axis_index("c")
        base = wid * b_per_w
        pltpu.sync_copy(idx_hbm.at[pl.ds(base, b_per_w)], idx_v)
        pltpu.async_copy(table_hbm.at[idx_v], rows_v, sem).wait()
        pltpu.sync_copy(rows_v, out_hbm.at[pl.ds(base, b_per_w)])

    return gather(codebook, indices)


def kernel(z_e, codebook):
    batch, num_codes, dim = z_e.shape
    n = batch * num_codes
    flat_z = z_e.reshape(n, dim)
    indices, acc = _argmin_distances(flat_z, codebook)
    zq_flat = _sc_gather(codebook, indices, n)
    vq_loss = _COMMIT * acc[0, 0] / jnp.float32(n * dim)
    return (zq_flat.reshape(batch, num_codes, dim), vq_loss,
            indices.reshape(batch, num_codes))


# speedup vs baseline: 1.1591x; 1.1591x over previous
import functools

import jax
import jax.numpy as jnp
from jax import lax
from jax.experimental import pallas as pl
from jax.experimental.pallas import tpu as pltpu
from jax.experimental.pallas import tpu_sc as plsc

_NUM_EMB = 8192
_DIM = 32
_ROW_BLOCK = 512
_COMMIT = 0.5


def _vq_body(z_ref, cb_ref, idx_ref, acc_ref, cbb_ref, cbsq_ref):
    i = pl.program_id(0)
    z = z_ref[...]

    @pl.when(i == 0)
    def _prep():
        cb = cb_ref[...]
        cbb_ref[...] = cb.astype(jnp.bfloat16)
        cbsq_ref[...] = jnp.sum(cb * cb, axis=1)[None, :]
        acc_ref[...] = jnp.zeros((1, 1), jnp.float32)

    z_sq = jnp.sum(z * z, axis=1, keepdims=True)
    cb_sq = cbsq_ref[...]
    s = jax.lax.dot_general(
        z.astype(jnp.bfloat16), cbb_ref[...],
        (((1,), (1,)), ((), ())),
        preferred_element_type=jnp.float32)
    dist = (z_sq + cb_sq) - 2.0 * s
    half = _NUM_EMB // 2
    d0 = dist[:, :half]
    d1 = dist[:, half:]
    m0 = jnp.min(d0, axis=1, keepdims=True)
    m1 = jnp.min(d1, axis=1, keepdims=True)
    fhalf = jnp.float32(half)
    colf = jax.lax.broadcasted_iota(
        jnp.int32, (1, half), 1).astype(jnp.float32)
    i0f = jnp.min(jnp.where(d0 == m0, colf, fhalf), axis=1)
    i1f = jnp.min(jnp.where(d1 == m1, colf, fhalf), axis=1) + fhalf
    m0_bf = m0[:, 0].astype(jnp.bfloat16).astype(jnp.float32)
    use1 = m1[:, 0] < m0_bf
    idx_ref[0, 0, :] = jnp.where(use1, i1f, i0f).astype(jnp.int32)

    m_sel = jnp.where(use1, m1[:, 0], m0[:, 0])
    acc_ref[...] += jnp.sum(m_sel).reshape(1, 1)


def _argmin_distances(flat_z, codebook):
    n = flat_z.shape[0]
    nblocks = n // _ROW_BLOCK
    idx3, acc = pl.pallas_call(
        _vq_body,
        grid=(nblocks,),
        in_specs=[
            pl.BlockSpec((_ROW_BLOCK, _DIM), lambda i: (i, 0)),
            pl.BlockSpec((_NUM_EMB, _DIM), lambda i: (0, 0)),
        ],
        out_specs=[
            pl.BlockSpec((1, 1, _ROW_BLOCK), lambda i: (i, 0, 0)),
            pl.BlockSpec((1, 1), lambda i: (0, 0)),
        ],
        out_shape=[
            jax.ShapeDtypeStruct((nblocks, 1, _ROW_BLOCK), jnp.int32),
            jax.ShapeDtypeStruct((1, 1), jnp.float32),
        ],
        scratch_shapes=[
            pltpu.VMEM((_NUM_EMB, _DIM), jnp.bfloat16),
            pltpu.VMEM((1, _NUM_EMB), jnp.float32),
        ],
        compiler_params=pltpu.CompilerParams(
            dimension_semantics=("arbitrary",)),
    )(flat_z, codebook)
    return idx3.reshape(n), acc


def _sc_gather(codebook, indices, n):
    info = plsc.get_sparse_core_info()
    nw = info.num_cores * info.num_subcores
    b_per_w = n // nw
    mesh = plsc.VectorSubcoreMesh(core_axis_name="c", subcore_axis_name="s")

    @functools.partial(
        pl.kernel, mesh=mesh,
        out_type=jax.ShapeDtypeStruct((n, _DIM), jnp.float32),
        scratch_types=[
            pltpu.VMEM((b_per_w,), jnp.int32),
            pltpu.VMEM((b_per_w, _DIM), jnp.float32),
            pltpu.SemaphoreType.DMA,
        ],
        compiler_params=pltpu.CompilerParams(use_tc_tiling_on_sc=False),
    )
    def gather(table_hbm, idx_hbm, out_hbm, idx_v, rows_v, sem):
        wid = lax.axis_index("s") * info.num_cores + lax.axis_index("c")
        base = wid * b_per_w
        pltpu.sync_copy(idx_hbm.at[pl.ds(base, b_per_w)], idx_v)
        pltpu.async_copy(table_hbm.at[idx_v], rows_v, sem).wait()
        pltpu.sync_copy(rows_v, out_hbm.at[pl.ds(base, b_per_w)])

    return gather(codebook, indices)


def kernel(z_e, codebook):
    batch, num_codes, dim = z_e.shape
    n = batch * num_codes
    flat_z = z_e.reshape(n, dim)
    indices, acc = _argmin_distances(flat_z, codebook)
    zq_flat = _sc_gather(codebook, indices, n)
    vq_loss = _COMMIT * acc[0, 0] / jnp.float32(n * dim)
    return (zq_flat.reshape(batch, num_codes, dim), vq_loss,
            indices.reshape(batch, num_codes))
